# baseline (speedup 1.0000x reference)
import functools

import jax
import jax.numpy as jnp
from jax import lax
from jax.experimental import pallas as pl
from jax.experimental.pallas import tpu as pltpu
from jax.experimental.pallas import tpu_sc as plsc

_VOCAB = 1000000
_DIM = 64
_B = 16384
_NC, _NS = 2, 16
_NW = _NC * _NS
_BPW = _B // _NW
_CH = 128
_NCHUNK = _BPW // _CH
_LANES = 16
_OROW = _BPW * _LANES // 128


def _sc_partials(center2d, pos2d, neg2d, in_pairs, out_pairs):
    mesh = plsc.VectorSubcoreMesh(core_axis_name="c", subcore_axis_name="s")

    @functools.partial(
        pl.kernel,
        out_type=[
            jax.ShapeDtypeStruct((_B * _LANES // 128, 128), jnp.float32),
            jax.ShapeDtypeStruct((_B * _LANES // 128, 128), jnp.float32),
        ],
        mesh=mesh,
        compiler_params=pltpu.CompilerParams(needs_layout_passes=False),
        scratch_types=[
            pltpu.VMEM((_BPW,), jnp.int32),
            pltpu.VMEM((_BPW,), jnp.int32),
            pltpu.VMEM((_BPW,), jnp.int32),
            pltpu.VMEM((_BPW,), jnp.int32),
            pltpu.VMEM((_BPW,), jnp.int32),
            pltpu.VMEM((_BPW,), jnp.int32),
            pltpu.VMEM((2 * _CH, 128), jnp.float32),
            pltpu.VMEM((2 * _CH, 128), jnp.float32),
            pltpu.VMEM((2 * _CH, 128), jnp.float32),
            pltpu.VMEM((_OROW, 128), jnp.float32),
            pltpu.VMEM((_OROW, 128), jnp.float32),
            pltpu.SemaphoreType.DMA,
            pltpu.SemaphoreType.DMA,
        ],
    )
    def k(center_hbm, pos_hbm, neg_hbm, in_emb, out_emb, opos_hbm, oneg_hbm,
          cids, pids, nids, cpair, ppair, npair,
          vbuf, pbuf, nbuf, tpos, tneg, sem0, sem1):
        wid = lax.axis_index("s") * _NC + lax.axis_index("c")
        sems = [sem0, sem1]

        for j in range(_NCHUNK):
            row = wid * _NCHUNK + j
            sl = pl.ds(j * _CH, _CH)
            pltpu.sync_copy(center_hbm.at[row], cids.at[sl])
            pltpu.sync_copy(pos_hbm.at[row], pids.at[sl])
            pltpu.sync_copy(neg_hbm.at[row], nids.at[sl])

        def shift_all(src, dst):
            def sbody(g, carry):
                sl = pl.ds(g * _LANES, _LANES)
                dst[sl] = jax.lax.shift_right_logical(src[sl], jnp.int32(1))
                return carry

            lax.fori_loop(0, _BPW // _LANES, sbody, 0)

        shift_all(cids, cpair)
        shift_all(pids, ppair)
        shift_all(nids, npair)

        def fire(j, slot):
            sl = pl.ds(j * _CH, _CH)
            dsl = pl.ds(slot * _CH, _CH)
            return [
                pltpu.async_copy(in_emb.at[cpair.at[sl]], vbuf.at[dsl],
                                 sems[slot]),
                pltpu.async_copy(out_emb.at[ppair.at[sl]], pbuf.at[dsl],
                                 sems[slot]),
                pltpu.async_copy(out_emb.at[npair.at[sl]], nbuf.at[dsl],
                                 sems[slot]),
            ]

        def scalar_at(ref, i):
            lane = jnp.full((_LANES,), i, jnp.int32)
            return jnp.max(plsc.load_gather(ref, [lane]))

        def compute(j, slot):
            def body(i, carry):
                r = slot * _CH + i
                g = j * _CH + i
                co = (scalar_at(cids, g) & 1) * _DIM
                po = (scalar_at(pids, g) & 1) * _DIM
                no = (scalar_at(nids, g) & 1) * _DIM
                v = [vbuf[r, pl.ds(co + m * _LANES, _LANES)] for m in range(4)]
                p = [pbuf[r, pl.ds(po + m * _LANES, _LANES)] for m in range(4)]
                n = [nbuf[r, pl.ds(no + m * _LANES, _LANES)] for m in range(4)]
                ap = v[0] * p[0] + v[1] * p[1] + v[2] * p[2] + v[3] * p[3]
                an = v[0] * n[0] + v[1] * n[1] + v[2] * n[2] + v[3] * n[3]
                flat = g * _LANES
                rr = flat // 128
                cc = flat % 128
                tpos[rr, pl.ds(cc, _LANES)] = ap
                tneg[rr, pl.ds(cc, _LANES)] = an
                return carry

            lax.fori_loop(0, _CH, body, 0)

        pend = {0: fire(0, 0)}
        for j in range(_NCHUNK):
            slot = j % 2
            if j + 1 < _NCHUNK:
                pend[(j + 1) % 2] = fire(j + 1, (j + 1) % 2)
            for d in pend[slot]:
                d.wait()
            compute(j, slot)

        base = wid * _OROW
        pltpu.sync_copy(tpos, opos_hbm.at[pl.ds(base, _OROW)])
        pltpu.sync_copy(tneg, oneg_hbm.at[pl.ds(base, _OROW)])

    return k(center2d, pos2d, neg2d, in_pairs, out_pairs)


def _tc_loss(tpos, tneg):
    def body(p_ref, n_ref, o_ref):
        gk = jax.lax.broadcasted_iota(jnp.int32, (128, 8), 0) // _LANES
        gg = jax.lax.broadcasted_iota(jnp.int32, (128, 8), 1)
        g = (gk == gg).astype(jnp.float32)
        ps = jnp.dot(p_ref[...], g, preferred_element_type=jnp.float32)
        ns = jnp.dot(n_ref[...], g, preferred_element_type=jnp.float32)
        sp = jax.nn.sigmoid(ps)
        sn = jax.nn.sigmoid(ns)
        loss = (-jnp.mean(jnp.log(sp + 1e-09))
                - jnp.mean(jnp.log(1.0 - sn + 1e-09)))
        o_ref[...] = jnp.broadcast_to(loss, (1, 1))

    out = pl.pallas_call(
        body,
        out_shape=jax.ShapeDtypeStruct((1, 1), jnp.float32),
    )(tpos, tneg)
    return out[0, 0]


_RBLK = 8192


def _tc_repack(table):
    def body(i_ref, o_ref):
        o_ref[...] = i_ref[...].reshape(_RBLK // 2, 2 * _DIM)

    return pl.pallas_call(
        body,
        grid=(_VOCAB // _RBLK,),
        in_specs=[pl.BlockSpec((_RBLK, _DIM), lambda i: (i, 0))],
        out_specs=pl.BlockSpec((_RBLK // 2, 2 * _DIM), lambda i: (i, 0)),
        out_shape=jax.ShapeDtypeStruct((_VOCAB // 2, 2 * _DIM), jnp.float32),
    )(table)


def kernel(center, pos, neg, input_emb, output_emb):
    center2d = center.astype(jnp.int32).reshape(_NW * _NCHUNK, _CH)
    pos2d = pos.astype(jnp.int32).reshape(_NW * _NCHUNK, _CH)
    neg2d = neg.astype(jnp.int32).reshape(_NW * _NCHUNK, _CH)
    in_pairs = jnp.concatenate([input_emb[0::2], input_emb[1::2]], axis=1)
    out_pairs = jnp.concatenate([output_emb[0::2], output_emb[1::2]], axis=1)
    tpos, tneg = _sc_partials(center2d, pos2d, neg2d, in_pairs, out_pairs)
    return _tc_loss(tpos, tneg)


# speedup vs baseline: 13.9025x; 13.9025x over previous
import functools

import jax
import jax.numpy as jnp
from jax import lax
from jax.experimental import pallas as pl
from jax.experimental.pallas import tpu as pltpu
from jax.experimental.pallas import tpu_sc as plsc

_VOCAB = 1000000
_DIM = 64
_B = 16384
_NC, _NS = 2, 16
_NW = _NC * _NS
_BPW = _B // _NW
_CH = 128
_NCHUNK = _BPW // _CH
_LANES = 16
_OROW = _BPW * _LANES // 128
_RBLK = 8192
_RSUB = 256


def _tc_repack(table):
    def body(i_ref, o_ref):
        r_id = jax.lax.broadcasted_iota(jnp.int32, (_RSUB // 2, _RSUB), 0)
        c_id = jax.lax.broadcasted_iota(jnp.int32, (_RSUB // 2, _RSUB), 1)
        ev = (c_id == 2 * r_id).astype(jnp.bfloat16)
        od = (c_id == 2 * r_id + 1).astype(jnp.bfloat16)
        for c in range(_RBLK // _RSUB):
            xc = i_ref[pl.ds(c * _RSUB, _RSUB), :].astype(jnp.bfloat16)
            e = jnp.dot(ev, xc, preferred_element_type=jnp.float32)
            o = jnp.dot(od, xc, preferred_element_type=jnp.float32)
            half = _RSUB // 2
            o_ref[pl.ds(c * half, half), pl.ds(0, _DIM)] = e
            o_ref[pl.ds(c * half, half), pl.ds(_DIM, _DIM)] = o

    return pl.pallas_call(
        body,
        grid=(_VOCAB // _RBLK,),
        in_specs=[pl.BlockSpec((_RBLK, _DIM), lambda i: (i, 0))],
        out_specs=pl.BlockSpec((_RBLK // 2, 2 * _DIM), lambda i: (i, 0)),
        out_shape=jax.ShapeDtypeStruct((_VOCAB // 2, 2 * _DIM), jnp.float32),
    )(table)


def _sc_partials(center2d, pos2d, neg2d, in_pairs, out_pairs):
    mesh = plsc.VectorSubcoreMesh(core_axis_name="c", subcore_axis_name="s")

    @functools.partial(
        pl.kernel,
        out_type=[
            jax.ShapeDtypeStruct((_B * _LANES // 128, 128), jnp.float32),
            jax.ShapeDtypeStruct((_B * _LANES // 128, 128), jnp.float32),
        ],
        mesh=mesh,
        compiler_params=pltpu.CompilerParams(needs_layout_passes=False),
        scratch_types=[
            pltpu.VMEM((_BPW,), jnp.int32),
            pltpu.VMEM((_BPW,), jnp.int32),
            pltpu.VMEM((_BPW,), jnp.int32),
            pltpu.VMEM((_BPW,), jnp.int32),
            pltpu.VMEM((_BPW,), jnp.int32),
            pltpu.VMEM((_BPW,), jnp.int32),
            pltpu.VMEM((2 * _CH, 128), jnp.float32),
            pltpu.VMEM((2 * _CH, 128), jnp.float32),
            pltpu.VMEM((2 * _CH, 128), jnp.float32),
            pltpu.VMEM((_OROW, 128), jnp.float32),
            pltpu.VMEM((_OROW, 128), jnp.float32),
            pltpu.SemaphoreType.DMA,
            pltpu.SemaphoreType.DMA,
        ],
    )
    def k(center_hbm, pos_hbm, neg_hbm, in_emb, out_emb, opos_hbm, oneg_hbm,
          cids, pids, nids, cpair, ppair, npair,
          vbuf, pbuf, nbuf, tpos, tneg, sem0, sem1):
        wid = lax.axis_index("s") * _NC + lax.axis_index("c")
        sems = [sem0, sem1]

        for j in range(_NCHUNK):
            row = wid * _NCHUNK + j
            sl = pl.ds(j * _CH, _CH)
            pltpu.sync_copy(center_hbm.at[row], cids.at[sl])
            pltpu.sync_copy(pos_hbm.at[row], pids.at[sl])
            pltpu.sync_copy(neg_hbm.at[row], nids.at[sl])

        def shift_all(src, dst):
            def sbody(g, carry):
                sl = pl.ds(g * _LANES, _LANES)
                dst[sl] = jax.lax.shift_right_logical(src[sl], jnp.int32(1))
                return carry

            lax.fori_loop(0, _BPW // _LANES, sbody, 0)

        shift_all(cids, cpair)
        shift_all(pids, ppair)
        shift_all(nids, npair)

        def fire(j, slot):
            sl = pl.ds(j * _CH, _CH)
            dsl = pl.ds(slot * _CH, _CH)
            return [
                pltpu.async_copy(in_emb.at[cpair.at[sl]], vbuf.at[dsl],
                                 sems[slot]),
                pltpu.async_copy(out_emb.at[ppair.at[sl]], pbuf.at[dsl],
                                 sems[slot]),
                pltpu.async_copy(out_emb.at[npair.at[sl]], nbuf.at[dsl],
                                 sems[slot]),
            ]

        def scalar_at(ref, i):
            lane = jnp.full((_LANES,), i, jnp.int32)
            return jnp.max(plsc.load_gather(ref, [lane]))

        def compute(j, slot):
            def body(i, carry):
                r = slot * _CH + i
                g = j * _CH + i
                co = (scalar_at(cids, g) & 1) * _DIM
                po = (scalar_at(pids, g) & 1) * _DIM
                no = (scalar_at(nids, g) & 1) * _DIM
                v = [vbuf[r, pl.ds(co + m * _LANES, _LANES)] for m in range(4)]
                p = [pbuf[r, pl.ds(po + m * _LANES, _LANES)] for m in range(4)]
                n = [nbuf[r, pl.ds(no + m * _LANES, _LANES)] for m in range(4)]
                ap = v[0] * p[0] + v[1] * p[1] + v[2] * p[2] + v[3] * p[3]
                an = v[0] * n[0] + v[1] * n[1] + v[2] * n[2] + v[3] * n[3]
                flat = g * _LANES
                rr = flat // 128
                cc = flat % 128
                tpos[rr, pl.ds(cc, _LANES)] = ap
                tneg[rr, pl.ds(cc, _LANES)] = an
                return carry

            lax.fori_loop(0, _CH, body, 0)

        pend = {0: fire(0, 0)}
        for j in range(_NCHUNK):
            slot = j % 2
            if j + 1 < _NCHUNK:
                pend[(j + 1) % 2] = fire(j + 1, (j + 1) % 2)
            for d in pend[slot]:
                d.wait()
            compute(j, slot)

        base = wid * _OROW
        pltpu.sync_copy(tpos, opos_hbm.at[pl.ds(base, _OROW)])
        pltpu.sync_copy(tneg, oneg_hbm.at[pl.ds(base, _OROW)])

    return k(center2d, pos2d, neg2d, in_pairs, out_pairs)


def _tc_loss(tpos, tneg):
    def body(p_ref, n_ref, o_ref):
        gk = jax.lax.broadcasted_iota(jnp.int32, (128, 8), 0) // _LANES
        gg = jax.lax.broadcasted_iota(jnp.int32, (128, 8), 1)
        g = (gk == gg).astype(jnp.float32)
        ps = jnp.dot(p_ref[...], g, preferred_element_type=jnp.float32)
        ns = jnp.dot(n_ref[...], g, preferred_element_type=jnp.float32)
        sp = jax.nn.sigmoid(ps)
        sn = jax.nn.sigmoid(ns)
        loss = (-jnp.mean(jnp.log(sp + 1e-09))
                - jnp.mean(jnp.log(1.0 - sn + 1e-09)))
        o_ref[...] = jnp.broadcast_to(loss, (1, 1))

    out = pl.pallas_call(
        body,
        out_shape=jax.ShapeDtypeStruct((1, 1), jnp.float32),
    )(tpos, tneg)
    return out[0, 0]


def kernel(center, pos, neg, input_emb, output_emb):
    center2d = center.astype(jnp.int32).reshape(_NW * _NCHUNK, _CH)
    pos2d = pos.astype(jnp.int32).reshape(_NW * _NCHUNK, _CH)
    neg2d = neg.astype(jnp.int32).reshape(_NW * _NCHUNK, _CH)
    in_pairs = _tc_repack(input_emb)
    out_pairs = _tc_repack(output_emb)
    tpos, tneg = _sc_partials(center2d, pos2d, neg2d, in_pairs, out_pairs)
    return _tc_loss(tpos, tneg)


# speedup vs baseline: 24.3286x; 1.7499x over previous
import functools

import jax
import jax.numpy as jnp
from jax import lax
from jax.experimental import pallas as pl
from jax.experimental.pallas import tpu as pltpu
from jax.experimental.pallas import tpu_sc as plsc

_VOCAB = 1000000
_DIM = 64
_B = 16384
_NC, _NS = 2, 16
_NW = _NC * _NS
_BPW = _B // _NW
_CH = 128
_NCHUNK = _BPW // _CH
_LANES = 16
_OROW = _BPW * _LANES // 128


def _sc_partials(center2d, pos2d, neg2d, input_emb, output_emb):
    mesh = plsc.VectorSubcoreMesh(core_axis_name="c", subcore_axis_name="s")

    @functools.partial(
        pl.kernel,
        out_type=[
            jax.ShapeDtypeStruct((_B * _LANES // 128, 128), jnp.float32),
            jax.ShapeDtypeStruct((_B * _LANES // 128, 128), jnp.float32),
        ],
        mesh=mesh,
        compiler_params=pltpu.CompilerParams(needs_layout_passes=False),
        scratch_types=[
            pltpu.VMEM((2 * _CH,), jnp.int32),
            pltpu.VMEM((2 * _CH,), jnp.int32),
            pltpu.VMEM((2 * _CH,), jnp.int32),
            pltpu.VMEM((2 * _CH, _DIM), jnp.float32),
            pltpu.VMEM((2 * _CH, _DIM), jnp.float32),
            pltpu.VMEM((2 * _CH, _DIM), jnp.float32),
            pltpu.VMEM((_OROW, 128), jnp.float32),
            pltpu.VMEM((_OROW, 128), jnp.float32),
            pltpu.SemaphoreType.DMA,
            pltpu.SemaphoreType.DMA,
            pltpu.SemaphoreType.DMA,
            pltpu.SemaphoreType.DMA,
            pltpu.SemaphoreType.DMA,
            pltpu.SemaphoreType.DMA,
        ],
    )
    def k(center_hbm, pos_hbm, neg_hbm, in_emb, out_emb, dummy_hbm,
          opos_hbm, oneg_hbm,
          cids, pids, nids, vbuf, pbuf, nbuf, tpos, tneg,
          sem0, sem1, sem2, sem3, sem4, sem5):
        wid = lax.axis_index("s") * _NC + lax.axis_index("c")
        sems = [[sem0, sem1, sem2], [sem3, sem4, sem5]]

        def stage(j, slot):
            row = wid * _NCHUNK + j
            sl = pl.ds(slot * _CH, _CH)
            pltpu.sync_copy(center_hbm.at[row], cids.at[sl])
            pltpu.sync_copy(pos_hbm.at[row], pids.at[sl])
            pltpu.sync_copy(neg_hbm.at[row], nids.at[sl])

        def scalar_at(ref, i):
            lane = jnp.full((_LANES,), i, jnp.int32)
            return jnp.max(plsc.load_gather(ref, [lane]))

        def fire(slot):
            def fbody(i, carry):
                r = slot * _CH + i
                ci = scalar_at(cids, r)
                pi = scalar_at(pids, r)
                ni = scalar_at(nids, r)
                pltpu.async_copy(in_emb.at[ci], vbuf.at[r], sems[slot][0])
                pltpu.async_copy(out_emb.at[pi], pbuf.at[r], sems[slot][1])
                pltpu.async_copy(out_emb.at[ni], nbuf.at[r], sems[slot][2])
                return carry

            lax.fori_loop(0, _CH, fbody, 0)

        def drain(slot):
            def dbody(i, carry):
                r = slot * _CH + i
                pltpu.make_async_copy(dummy_hbm, vbuf.at[r], sems[slot][0]).wait()
                pltpu.make_async_copy(dummy_hbm, pbuf.at[r], sems[slot][1]).wait()
                pltpu.make_async_copy(dummy_hbm, nbuf.at[r], sems[slot][2]).wait()
                return carry

            lax.fori_loop(0, _CH, dbody, 0)

        def compute(j, slot):
            def body(i, carry):
                r = slot * _CH + i
                v = [vbuf[r, pl.ds(m * _LANES, _LANES)] for m in range(4)]
                p = [pbuf[r, pl.ds(m * _LANES, _LANES)] for m in range(4)]
                n = [nbuf[r, pl.ds(m * _LANES, _LANES)] for m in range(4)]
                ap = v[0] * p[0] + v[1] * p[1] + v[2] * p[2] + v[3] * p[3]
                an = v[0] * n[0] + v[1] * n[1] + v[2] * n[2] + v[3] * n[3]
                flat = (j * _CH + i) * _LANES
                r = flat // 128
                c = flat % 128
                tpos[r, pl.ds(c, _LANES)] = ap
                tneg[r, pl.ds(c, _LANES)] = an
                return carry

            lax.fori_loop(0, _CH, body, 0)

        stage(0, 0)
        fire(0)
        for j in range(_NCHUNK):
            slot = j % 2
            if j + 1 < _NCHUNK:
                stage(j + 1, (j + 1) % 2)
                fire((j + 1) % 2)
            drain(slot)
            compute(j, slot)

        base = wid * _OROW
        pltpu.sync_copy(tpos, opos_hbm.at[pl.ds(base, _OROW)])
        pltpu.sync_copy(tneg, oneg_hbm.at[pl.ds(base, _OROW)])

    dummy = jnp.zeros((_DIM,), jnp.float32)
    return k(center2d, pos2d, neg2d, input_emb, output_emb, dummy)


def _tc_loss(tpos, tneg):
    def body(p_ref, n_ref, o_ref):
        gk = jax.lax.broadcasted_iota(jnp.int32, (128, 8), 0) // _LANES
        gg = jax.lax.broadcasted_iota(jnp.int32, (128, 8), 1)
        g = (gk == gg).astype(jnp.float32)
        ps = jnp.dot(p_ref[...], g, preferred_element_type=jnp.float32)
        ns = jnp.dot(n_ref[...], g, preferred_element_type=jnp.float32)
        sp = jax.nn.sigmoid(ps)
        sn = jax.nn.sigmoid(ns)
        loss = (-jnp.mean(jnp.log(sp + 1e-09))
                - jnp.mean(jnp.log(1.0 - sn + 1e-09)))
        o_ref[...] = jnp.broadcast_to(loss, (1, 1))

    out = pl.pallas_call(
        body,
        out_shape=jax.ShapeDtypeStruct((1, 1), jnp.float32),
    )(tpos, tneg)
    return out[0, 0]


def kernel(center, pos, neg, input_emb, output_emb):
    center2d = center.astype(jnp.int32).reshape(_NW * _NCHUNK, _CH)
    pos2d = pos.astype(jnp.int32).reshape(_NW * _NCHUNK, _CH)
    neg2d = neg.astype(jnp.int32).reshape(_NW * _NCHUNK, _CH)
    tpos, tneg = _sc_partials(center2d, pos2d, neg2d, input_emb, output_emb)
    return _tc_loss(tpos, tneg)
